# baseline (speedup 1.0000x reference)
import jax
import jax.numpy as jnp
from jax.experimental import pallas as pl
from jax.experimental.pallas import tpu as pltpu

_BM = 200
_NSTREAMS = 2


def _gnn_body(msteps, bm, n, nstreams, refs):
    adj_refs = refs[:nstreams]
    (feat_ref, w_ref, gamma_ref, beta_ref,
     out_ref, support_ref, stats_ref) = refs[nstreams:]
    i = pl.program_id(0)

    @pl.when(i == 0)
    def _init():
        support_ref[...] = jnp.dot(feat_ref[...], w_ref[...],
                                   preferred_element_type=jnp.float32)
        stats_ref[...] = jnp.zeros_like(stats_ref)

    @pl.when(i < msteps)
    def _compute():
        s0 = jnp.zeros((1, out_ref.shape[1]), jnp.float32)
        s1 = jnp.zeros((1, out_ref.shape[1]), jnp.float32)
        for s, adj_ref in enumerate(adj_refs):
            z = jnp.dot(adj_ref[...], support_ref[...],
                        preferred_element_type=jnp.float32)
            out_ref[pl.ds((nstreams * i + s) * bm, bm), :] = z
            s0 += jnp.sum(z, axis=0, keepdims=True)
            s1 += jnp.sum(z * z, axis=0, keepdims=True)
        stats_ref[0:1, :] += s0
        stats_ref[1:2, :] += s1

    @pl.when(i == msteps)
    def _normalize():
        mean = stats_ref[0:1, :] / n
        var = stats_ref[1:2, :] / n - mean * mean
        inv = jax.lax.rsqrt(var + 1e-5)
        scale = inv * gamma_ref[...]
        shift = beta_ref[...] - mean * scale
        cm = 400 if n % 400 == 0 else bm
        csteps = n // cm

        def body(j, _):
            blk = out_ref[pl.ds(j * cm, cm), :]
            out_ref[pl.ds(j * cm, cm), :] = jnp.maximum(blk * scale + shift,
                                                        0.0)
            return 0

        jax.lax.fori_loop(0, csteps, body, 0)


def kernel(features, adj, weight, bias, gamma, beta):
    del bias
    n, in_dim = features.shape
    out_dim = weight.shape[1]
    bm, nstreams = _BM, _NSTREAMS
    if n % (nstreams * bm) != 0:
        nstreams = 2 if n % (2 * bm) == 0 else 1
    msteps = n // (nstreams * bm)

    gamma2 = gamma.reshape(1, out_dim)
    beta2 = beta.reshape(1, out_dim)

    def body(*refs):
        _gnn_body(msteps, bm, n, nstreams, refs)

    last = msteps - 1
    adj_specs = [
        pl.BlockSpec((bm, n),
                     lambda i, s=s: (nstreams * jnp.minimum(i, last) + s, 0))
        for s in range(nstreams)
    ]
    return pl.pallas_call(
        body,
        grid=(msteps + 1,),
        in_specs=adj_specs + [
            pl.BlockSpec((n, in_dim), lambda i: (0, 0)),
            pl.BlockSpec((in_dim, out_dim), lambda i: (0, 0)),
            pl.BlockSpec((1, out_dim), lambda i: (0, 0)),
            pl.BlockSpec((1, out_dim), lambda i: (0, 0)),
        ],
        out_specs=pl.BlockSpec((n, out_dim), lambda i: (0, 0)),
        out_shape=jax.ShapeDtypeStruct((n, out_dim), jnp.float32),
        scratch_shapes=[
            pltpu.VMEM((n, out_dim), jnp.float32),
            pltpu.VMEM((8, out_dim), jnp.float32),
        ],
        compiler_params=pltpu.CompilerParams(
            dimension_semantics=("arbitrary",),
            vmem_limit_bytes=60 * 1024 * 1024,
        ),
    )(*([adj] * nstreams), features, weight, gamma2, beta2)
